# Initial kernel scaffold; baseline (speedup 1.0000x reference)
import jax
import jax.numpy as jnp
from jax.experimental import pallas as pl


def kernel(W, beta, H, pred, Y, particle_id, track_params, reconstructable):
    raise NotImplementedError("write your pallas kernel here")


# speedup vs baseline: 1.6602x; 1.6602x over previous
import functools

import jax
import jax.numpy as jnp
from jax import lax
from jax.experimental import pallas as pl
from jax.experimental.pallas import tpu as pltpu
from jax.experimental.pallas import tpu_sc as plsc

N = 500000
D = 5
P = 1000
P2 = 1024
SCALE = 100.0

NC = 2
NS = 16
NW = NC * NS

CHUNK = 2000
NCHUNKS = N // CHUNK
GROUPS = CHUNK // 16
FULL = NCHUNKS // NW
EXTRA = NCHUNKS % NW

_mesh = plsc.VectorSubcoreMesh(core_axis_name="c", subcore_axis_name="s")


@functools.partial(
    pl.kernel,
    mesh=_mesh,
    compiler_params=pltpu.CompilerParams(needs_layout_passes=False),
    out_type=[
        jax.ShapeDtypeStruct((NW, P2), jnp.float32),
        jax.ShapeDtypeStruct((NW, P2), jnp.float32),
    ],
    scratch_types=[
        pltpu.VMEM((CHUNK * D,), jnp.float32),
        pltpu.VMEM((CHUNK * D,), jnp.float32),
        pltpu.VMEM((CHUNK,), jnp.int32),
        pltpu.VMEM((CHUNK,), jnp.int32),
        pltpu.VMEM((P2,), jnp.float32),
        pltpu.VMEM((P2,), jnp.float32),
    ],
)
def _sc_accum(pred_hbm, tp_hbm, pid_hbm, rec_hbm, sums_out, cnts_out,
              pbuf, tbuf, pidbuf, recbuf, sums, cnts):
    wid = lax.axis_index("c") * NS + lax.axis_index("s")

    zero16 = jnp.zeros((16,), jnp.float32)

    def _zero_body(i, _):
        sums[pl.ds(i * 16, 16)] = zero16
        cnts[pl.ds(i * 16, 16)] = zero16
        return 0

    lax.fori_loop(0, P2 // 16, _zero_body, 0)

    lanes = lax.iota(jnp.int32, 16)

    def _do_chunk(c):
        base = c * CHUNK
        pltpu.sync_copy(pred_hbm.at[pl.ds(base * D, CHUNK * D)], pbuf)
        pltpu.sync_copy(tp_hbm.at[pl.ds(base * D, CHUNK * D)], tbuf)
        pltpu.sync_copy(pid_hbm.at[pl.ds(base, CHUNK)], pidbuf)
        pltpu.sync_copy(rec_hbm.at[pl.ds(base, CHUNK)], recbuf)

        def _group_body(j, _):
            idx5 = (j * 16 + lanes) * D
            acc = jnp.zeros((16,), jnp.float32)
            for d in range(D):
                a = plsc.load_gather(pbuf, [idx5 + d])
                b = plsc.load_gather(tbuf, [idx5 + d])
                df = a - b
                acc = acc + df * df
            pid = pidbuf[pl.ds(j * 16, 16)]
            rec = recbuf[pl.ds(j * 16, 16)]
            valid = (rec > 0) & (pid > 0)
            pid_eff = jnp.where(valid, pid, 0)
            vf = valid.astype(jnp.float32)
            plsc.addupdate_scatter(sums, [pid_eff], acc * vf)
            plsc.addupdate_scatter(cnts, [pid_eff], vf)
            return 0

        lax.fori_loop(0, GROUPS, _group_body, 0)

    def _chunk_body(ci, _):
        _do_chunk(wid + ci * NW)
        return 0

    lax.fori_loop(0, FULL, _chunk_body, 0)

    @pl.when(wid < EXTRA)
    def _():
        _do_chunk(wid + FULL * NW)

    pltpu.sync_copy(sums, sums_out.at[wid])
    pltpu.sync_copy(cnts, cnts_out.at[wid])


def _finalize_body(sums_ref, cnts_ref, out_ref):
    s = jnp.sum(sums_ref[...], axis=0, keepdims=True)
    c = jnp.sum(cnts_ref[...], axis=0, keepdims=True)
    pid = lax.broadcasted_iota(jnp.int32, (1, P2), 1)
    present = (pid > 0) & (c > 0.0)
    denom = jnp.where(present, c, 1.0)
    terms = jnp.where(present, s / denom, 0.0)
    loss = jnp.sum(terms)
    k = jnp.sum(present.astype(jnp.float32))
    out_ref[...] = jnp.reshape(SCALE * loss / k, (1, 1))


def kernel(W, beta, H, pred, Y, particle_id, track_params, reconstructable):
    pred_flat = jnp.reshape(pred, (N * D,))
    tp_flat = jnp.reshape(track_params, (N * D,))
    sums, cnts = _sc_accum(pred_flat, tp_flat, particle_id, reconstructable)
    out = pl.pallas_call(
        _finalize_body,
        out_shape=jax.ShapeDtypeStruct((1, 1), jnp.float32),
    )(sums, cnts)
    return out[0, 0]
